# baseline (speedup 1.0000x reference)
import jax
import jax.numpy as jnp
from jax import lax
from jax.experimental import pallas as pl
from jax.experimental.pallas import tpu as pltpu
from jax.experimental.pallas import tpu_sc as plsc

BATCH = 4096
SEQ = 200
DIM = 64

_B = BATCH * SEQ
_NW = 32
_BPW = _B // _NW
_BB = 128
_SCS = 8
_NCH = SEQ // _SCS
_HROWS = _BB * 4


def _embed_kernel(idx_hbm, tab_hbm, out_hbm,
                  idx_v, cidx, par, buf, stage, gsem, wsem):
    nc = 2
    wid = lax.axis_index("s") * nc + lax.axis_index("c")
    base = wid * _BPW
    pltpu.sync_copy(idx_hbm.at[pl.ds(base, _BPW)], idx_v)

    iota = lax.iota(jnp.int32, 16)
    iota128 = iota * 128

    def compact(b, s0):
        v = idx_v[pl.ds(b * SEQ + s0, 16)]
        pos = iota128 + b
        plsc.store_scatter(cidx, [pos], lax.shift_right_logical(v, 1))
        plsc.store_scatter(par, [pos], lax.bitwise_and(v, 1))
        return s0

    def halftranspose(h):
        @plsc.parallel_loop(0, 256, unroll=8)
        def per_sdtb(i):
            si = i // 64
            dt = (i // 8) % 8
            bg = i % 8
            rbase = si * _BB + bg * 16
            row = iota + rbase
            pvec = par[pl.ds(h * _HROWS + rbase, 16)]
            col = pvec * 64 + (dt * 8)
            for d8 in range(8):
                v = plsc.load_gather(buf, [row, col + d8])
                stage[si, dt, d8, pl.ds(bg * 16, 16)] = v

    def body(g, carry):
        del carry
        s0 = g * _SCS
        lax.fori_loop(0, _BB, compact, s0)
        for h in range(2):
            pltpu.async_copy(
                tab_hbm.at[cidx.at[pl.ds(h * _HROWS, _HROWS)]], buf, gsem)
            pltpu.make_async_copy(
                tab_hbm.at[cidx.at[pl.ds(h * _HROWS, _HROWS)]], buf, gsem).wait()
            @pl.when(g * 2 + h >= 1)
            def _():
                pltpu.make_async_copy(
                    stage, out_hbm.at[pl.ds(0, 4), :, wid], wsem).wait()
            halftranspose(h)
            pltpu.async_copy(
                stage, out_hbm.at[pl.ds(s0 + h * 4, 4), :, wid], wsem)
        return 0

    lax.fori_loop(0, _NCH, body, 0)
    pltpu.make_async_copy(
        stage, out_hbm.at[pl.ds(0, 4), :, wid], wsem).wait()


@jax.jit
def kernel(input_ids, embed_tokens):
    idx = input_ids.reshape(-1).astype(jnp.int32)
    tab = embed_tokens.reshape(500000, 128)
    mesh = plsc.VectorSubcoreMesh(core_axis_name="c", subcore_axis_name="s")
    out5 = pl.kernel(
        _embed_kernel,
        mesh=mesh,
        compiler_params=pltpu.CompilerParams(
            use_tc_tiling_on_sc=False, needs_layout_passes=False),
        out_type=jax.ShapeDtypeStruct((SEQ, 8, _NW, 8, 128), jnp.float32),
        scratch_types=[
            pltpu.VMEM((_BPW,), jnp.int32),
            pltpu.VMEM((2 * _BB * _SCS,), jnp.int32),
            pltpu.VMEM((2 * _BB * _SCS,), jnp.int32),
            pltpu.VMEM((_HROWS, 128), jnp.float32),
            pltpu.VMEM((4, 8, 8, 128), jnp.float32),
            pltpu.SemaphoreType.DMA,
            pltpu.SemaphoreType.DMA,
        ],
    )(idx, tab)
    return out5.transpose((2, 4, 0, 1, 3)).reshape(BATCH, SEQ, DIM)


# speedup vs baseline: 1.3333x; 1.3333x over previous
import jax
import jax.numpy as jnp
from jax import lax
from jax.experimental import pallas as pl
from jax.experimental.pallas import tpu as pltpu
from jax.experimental.pallas import tpu_sc as plsc

BATCH = 4096
SEQ = 200
DIM = 64

_B = BATCH * SEQ
_NW = 32
_BPW = _B // _NW
_NBUF = 4
_CHUNK = 400
_NCHUNK = _BPW // _CHUNK


def _embed_kernel(idx_hbm, table_hbm, out_hbm,
                  idx_v, b0, b1, b2, b3,
                  g0, g1, g2, g3, w0, w1, w2, w3):
    nc = 2
    wid = lax.axis_index("s") * nc + lax.axis_index("c")
    base = wid * _BPW
    pltpu.sync_copy(idx_hbm.at[pl.ds(base, _BPW)], idx_v)

    bufs = (b0, b1, b2, b3)
    gsems = (g0, g1, g2, g3)
    wsems = (w0, w1, w2, w3)

    def gather_start(g, b):
        pltpu.async_copy(
            table_hbm.at[idx_v.at[pl.ds(g * _CHUNK, _CHUNK)]], bufs[b],
            gsems[b])

    def gather_wait(b):
        pltpu.make_async_copy(
            out_hbm.at[pl.ds(base, _CHUNK)], bufs[b], gsems[b]).wait()

    def write_start(g, b):
        pltpu.async_copy(
            bufs[b], out_hbm.at[pl.ds(base + g * _CHUNK, _CHUNK)], wsems[b])

    def write_wait(b):
        pltpu.make_async_copy(
            bufs[b], out_hbm.at[pl.ds(base, _CHUNK)], wsems[b]).wait()

    for g in range(_NBUF - 1):
        gather_start(g, g)

    def body(i, carry):
        del carry
        for par in range(_NBUF):
            gg = i * _NBUF + par
            gather_wait(par)
            write_start(gg, par)
            nxt = (par + _NBUF - 1) % _NBUF
            @pl.when(jnp.logical_and(gg >= 1, gg + _NBUF - 1 < _NCHUNK))
            def _():
                write_wait(nxt)
            @pl.when(gg + _NBUF - 1 < _NCHUNK)
            def _():
                gather_start(gg + _NBUF - 1, nxt)
        return 0

    lax.fori_loop(0, _NCHUNK // _NBUF, body, 0)
    for b in range(_NBUF):
        write_wait(b)


@jax.jit
def kernel(input_ids, embed_tokens):
    idx = input_ids.reshape(-1).astype(jnp.int32)
    mesh = plsc.VectorSubcoreMesh(core_axis_name="c", subcore_axis_name="s")
    out = pl.kernel(
        _embed_kernel,
        mesh=mesh,
        compiler_params=pltpu.CompilerParams(use_tc_tiling_on_sc=False),
        out_type=jax.ShapeDtypeStruct((_B, DIM), jnp.float32),
        scratch_types=[
            pltpu.VMEM((_BPW,), jnp.int32),
            pltpu.VMEM((_CHUNK, DIM), jnp.float32),
            pltpu.VMEM((_CHUNK, DIM), jnp.float32),
            pltpu.VMEM((_CHUNK, DIM), jnp.float32),
            pltpu.VMEM((_CHUNK, DIM), jnp.float32),
            pltpu.SemaphoreType.DMA,
            pltpu.SemaphoreType.DMA,
            pltpu.SemaphoreType.DMA,
            pltpu.SemaphoreType.DMA,
            pltpu.SemaphoreType.DMA,
            pltpu.SemaphoreType.DMA,
            pltpu.SemaphoreType.DMA,
            pltpu.SemaphoreType.DMA,
        ],
    )(idx, embed_tokens)
    return out.reshape(BATCH, SEQ, DIM)
